# baseline (speedup 1.0000x reference)
import functools

import jax
import jax.numpy as jnp
from jax import lax
from jax.experimental import pallas as pl
from jax.experimental.pallas import tpu as pltpu
from jax.experimental.pallas import tpu_sc as plsc

_NUM_CORES = 2
_NUM_SUBCORES = 16
_NUM_WORKERS = _NUM_CORES * _NUM_SUBCORES
_LANES = 16


def _sc_pool_fn(batch, ctx, vocab, embed):
  bpw = batch // _NUM_WORKERS
  lanes_per_row = embed // _LANES
  inv_ctx = 1.0 / ctx

  def body(table_hbm, ids_hbm, out_hbm, raw_v, rows_a, rows_b, acc_v,
           sem_a, sem_b):
    wid = lax.axis_index("s") * _NUM_CORES + lax.axis_index("c")
    base = wid * bpw
    chunk = 128
    n_chunks = (bpw * ctx) // chunk
    pltpu.sync_copy(ids_hbm.at[pl.ds(wid * n_chunks, n_chunks)], raw_v)

    zeros = jnp.zeros((_LANES,), jnp.float32)

    def zero_row(r, carry):
      for c in range(lanes_per_row):
        acc_v[r, pl.ds(c * _LANES, _LANES)] = zeros
      return carry

    lax.fori_loop(0, bpw, zero_row, 0)

    bufs = (rows_a, rows_b)
    sems = (sem_a, sem_b)

    def gather(c):
      return pltpu.async_copy(
          table_hbm.at[raw_v.at[c]], bufs[c % 2], sems[c % 2])

    inflight = gather(0)
    for c in range(n_chunks):
      nxt = gather(c + 1) if c + 1 < n_chunks else None
      inflight.wait()
      buf = bufs[c % 2]
      p0 = c * chunk

      def add_row(k, carry, buf=buf, p0=p0):
        r = (p0 + k) // ctx
        for cc in range(lanes_per_row):
          sl = pl.ds(cc * _LANES, _LANES)
          acc_v[r, sl] = acc_v[r, sl] + buf[k, sl]
        return carry

      lax.fori_loop(0, chunk, add_row, 0)
      inflight = nxt

    def scale_row(r, carry):
      for c in range(lanes_per_row):
        sl = pl.ds(c * _LANES, _LANES)
        acc_v[r, sl] = acc_v[r, sl] * inv_ctx
      return carry

    lax.fori_loop(0, bpw, scale_row, 0)
    pltpu.sync_copy(acc_v, out_hbm.at[pl.ds(base, bpw)])

  return pl.kernel(
      body,
      out_type=jax.ShapeDtypeStruct((batch, embed), jnp.float32),
      mesh=plsc.VectorSubcoreMesh(core_axis_name="c", subcore_axis_name="s"),
      compiler_params=pltpu.CompilerParams(use_tc_tiling_on_sc=False),
      scratch_types=[
          pltpu.VMEM(((bpw * ctx) // 128, 128), jnp.int32),
          pltpu.VMEM((128, embed), jnp.float32),
          pltpu.VMEM((128, embed), jnp.float32),
          pltpu.VMEM((bpw, embed), jnp.float32),
          pltpu.SemaphoreType.DMA,
          pltpu.SemaphoreType.DMA,
      ],
  )


def _mm_body(p_ref, w_ref, b_ref, o_ref):
  o_ref[...] = lax.dot_general(
      p_ref[...], w_ref[...],
      dimension_numbers=(((1,), (1,)), ((), ())),
      preferred_element_type=jnp.float32) + b_ref[...]


def _mm_fn(batch, vocab, embed, n_tile):
  grid = (pl.cdiv(vocab, n_tile),)
  return pl.pallas_call(
      _mm_body,
      grid=grid,
      in_specs=[
          pl.BlockSpec((batch, embed), lambda i: (0, 0)),
          pl.BlockSpec((n_tile, embed), lambda i: (i, 0)),
          pl.BlockSpec((1, n_tile), lambda i: (0, i)),
      ],
      out_specs=pl.BlockSpec((batch, n_tile), lambda i: (0, i)),
      out_shape=jax.ShapeDtypeStruct((batch, vocab), jnp.float32),
  )


@functools.lru_cache(maxsize=None)
def _build(batch, ctx, vocab, embed):
  return _sc_pool_fn(batch, ctx, vocab, embed), _mm_fn(batch, vocab, embed, 512)


def kernel(context_ids, embed_table, fc_weight, fc_bias):
  batch, ctx = context_ids.shape
  vocab, embed = embed_table.shape
  sc_pool, mm = _build(batch, ctx, vocab, embed)
  ids_rows = context_ids.astype(jnp.int32).reshape((batch * ctx) // 128, 128)
  pooled = sc_pool(embed_table, ids_rows)
  return mm(pooled, fc_weight, fc_bias.reshape(1, vocab))


# speedup vs baseline: 1.0002x; 1.0002x over previous
import functools

import jax
import jax.numpy as jnp
from jax import lax
from jax.experimental import pallas as pl
from jax.experimental.pallas import tpu as pltpu
from jax.experimental.pallas import tpu_sc as plsc

_NUM_CORES = 2
_NUM_SUBCORES = 16
_NUM_WORKERS = _NUM_CORES * _NUM_SUBCORES
_LANES = 16


def _sc_pool_fn(batch, ctx, vocab, embed):
  bpw = batch // _NUM_WORKERS
  lanes_per_row = embed // _LANES
  inv_ctx = 1.0 / ctx

  def body(table_hbm, ids_hbm, out_hbm, raw_v, rows_a, rows_b, acc_v,
           sem_a, sem_b):
    wid = lax.axis_index("s") * _NUM_CORES + lax.axis_index("c")
    base = wid * bpw
    pltpu.sync_copy(ids_hbm.at[pl.ds(base, bpw)], raw_v)

    bufs = (rows_a, rows_b)
    sems = (sem_a, sem_b)

    def fire(r, b):
      return pltpu.async_copy(table_hbm.at[raw_v.at[r]], bufs[b], sems[b])

    def drain(b):
      pltpu.make_async_copy(
          table_hbm.at[pl.ds(0, ctx)], bufs[b], sems[b]).wait()

    fire(0, 0)
    fire(1, 1)

    def pair_body(i, carry):
      for b in range(2):
        r = 2 * i + b
        drain(b)
        buf = bufs[b]
        sums = [buf[0, pl.ds(cc * _LANES, _LANES)]
                for cc in range(lanes_per_row)]
        for t in range(1, ctx):
          for cc in range(lanes_per_row):
            sums[cc] = sums[cc] + buf[t, pl.ds(cc * _LANES, _LANES)]

        @pl.when(r + 2 < bpw)
        def _():
          fire(r + 2, b)

        for cc in range(lanes_per_row):
          acc_v[r, pl.ds(cc * _LANES, _LANES)] = sums[cc] * inv_ctx
      return carry

    lax.fori_loop(0, bpw // 2, pair_body, 0)
    pltpu.sync_copy(acc_v, out_hbm.at[pl.ds(base, bpw)])

  return pl.kernel(
      body,
      out_type=jax.ShapeDtypeStruct((batch, embed), jnp.float32),
      mesh=plsc.VectorSubcoreMesh(core_axis_name="c", subcore_axis_name="s"),
      compiler_params=pltpu.CompilerParams(use_tc_tiling_on_sc=False),
      scratch_types=[
          pltpu.VMEM((bpw, ctx), jnp.int32),
          pltpu.VMEM((ctx, embed), jnp.float32),
          pltpu.VMEM((ctx, embed), jnp.float32),
          pltpu.VMEM((bpw, embed), jnp.float32),
          pltpu.SemaphoreType.DMA,
          pltpu.SemaphoreType.DMA,
      ],
  )


def _mm_body(p_ref, w_ref, b_ref, o_ref):
  o_ref[...] = lax.dot_general(
      p_ref[...], w_ref[...],
      dimension_numbers=(((1,), (1,)), ((), ())),
      preferred_element_type=jnp.float32) + b_ref[...]


def _mm_fn(batch, vocab, embed, n_tile):
  grid = (pl.cdiv(vocab, n_tile),)
  return pl.pallas_call(
      _mm_body,
      grid=grid,
      in_specs=[
          pl.BlockSpec((batch, embed), lambda i: (0, 0)),
          pl.BlockSpec((n_tile, embed), lambda i: (i, 0)),
          pl.BlockSpec((1, n_tile), lambda i: (0, i)),
      ],
      out_specs=pl.BlockSpec((batch, n_tile), lambda i: (0, i)),
      out_shape=jax.ShapeDtypeStruct((batch, vocab), jnp.float32),
  )


@functools.lru_cache(maxsize=None)
def _build(batch, ctx, vocab, embed):
  return _sc_pool_fn(batch, ctx, vocab, embed), _mm_fn(batch, vocab, embed, 512)


def kernel(context_ids, embed_table, fc_weight, fc_bias):
  batch, ctx = context_ids.shape
  vocab, embed = embed_table.shape
  sc_pool, mm = _build(batch, ctx, vocab, embed)
  pooled = sc_pool(embed_table, context_ids.astype(jnp.int32))
  return mm(pooled, fc_weight, fc_bias.reshape(1, vocab))


# speedup vs baseline: 1.0023x; 1.0021x over previous
import functools

import jax
import jax.numpy as jnp
from jax import lax
from jax.experimental import pallas as pl
from jax.experimental.pallas import tpu as pltpu
from jax.experimental.pallas import tpu_sc as plsc

_NUM_CORES = 2
_NUM_SUBCORES = 16
_NUM_WORKERS = _NUM_CORES * _NUM_SUBCORES
_LANES = 16


def _sc_pool_fn(batch, ctx, vocab, embed):
  bpw = batch // _NUM_WORKERS
  lanes_per_row = embed // _LANES
  inv_ctx = 1.0 / ctx

  def body(table_hbm, ids_hbm, out_hbm, raw_v, rows_a, rows_b, acc_v,
           sem_a, sem_b):
    wid = lax.axis_index("s") * _NUM_CORES + lax.axis_index("c")
    base = wid * bpw
    pltpu.sync_copy(ids_hbm.at[pl.ds(base, bpw)], raw_v)

    bufs = (rows_a, rows_b)
    sems = (sem_a, sem_b)

    def fire(r, b):
      return pltpu.async_copy(table_hbm.at[raw_v.at[r]], bufs[b], sems[b])

    def drain(b):
      pltpu.make_async_copy(
          table_hbm.at[pl.ds(0, ctx)], bufs[b], sems[b]).wait()

    fire(0, 0)
    fire(1, 1)

    def pair_body(i, carry):
      for b in range(2):
        r = 2 * i + b
        drain(b)
        buf = bufs[b]
        sums = [buf[0, pl.ds(cc * _LANES, _LANES)]
                for cc in range(lanes_per_row)]
        for t in range(1, ctx):
          for cc in range(lanes_per_row):
            sums[cc] = sums[cc] + buf[t, pl.ds(cc * _LANES, _LANES)]

        @pl.when(r + 2 < bpw)
        def _():
          fire(r + 2, b)

        for cc in range(lanes_per_row):
          acc_v[r, pl.ds(cc * _LANES, _LANES)] = sums[cc] * inv_ctx
      return carry

    lax.fori_loop(0, bpw // 2, pair_body, 0)
    pltpu.sync_copy(acc_v, out_hbm.at[pl.ds(base, bpw)])

  return pl.kernel(
      body,
      out_type=jax.ShapeDtypeStruct((batch, embed), jnp.float32),
      mesh=plsc.VectorSubcoreMesh(core_axis_name="c", subcore_axis_name="s"),
      compiler_params=pltpu.CompilerParams(use_tc_tiling_on_sc=False),
      scratch_types=[
          pltpu.VMEM((bpw, ctx), jnp.int32),
          pltpu.VMEM((ctx, embed), jnp.float32),
          pltpu.VMEM((ctx, embed), jnp.float32),
          pltpu.VMEM((bpw, embed), jnp.float32),
          pltpu.SemaphoreType.DMA,
          pltpu.SemaphoreType.DMA,
      ],
  )


def _mm_body(p_ref, w_ref, b_ref, o_ref):
  o_ref[...] = lax.dot_general(
      p_ref[...], w_ref[...],
      dimension_numbers=(((1,), (1,)), ((), ())),
      preferred_element_type=jnp.float32) + b_ref[...]


def _mm_fn(batch, vocab, embed, n_tile):
  grid = (pl.cdiv(vocab, n_tile),)
  return pl.pallas_call(
      _mm_body,
      grid=grid,
      in_specs=[
          pl.BlockSpec((batch, embed), lambda i: (0, 0)),
          pl.BlockSpec((n_tile, embed), lambda i: (i, 0)),
          pl.BlockSpec((1, n_tile), lambda i: (0, i)),
      ],
      out_specs=pl.BlockSpec((batch, n_tile), lambda i: (0, i)),
      out_shape=jax.ShapeDtypeStruct((batch, vocab), jnp.float32),
  )


@functools.lru_cache(maxsize=None)
def _build(batch, ctx, vocab, embed):
  return _sc_pool_fn(batch, ctx, vocab, embed), _mm_fn(batch, vocab, embed, 1024)


def kernel(context_ids, embed_table, fc_weight, fc_bias):
  batch, ctx = context_ids.shape
  vocab, embed = embed_table.shape
  sc_pool, mm = _build(batch, ctx, vocab, embed)
  pooled = sc_pool(embed_table, context_ids.astype(jnp.int32))
  return mm(pooled, fc_weight, fc_bias.reshape(1, vocab))


# speedup vs baseline: 1.0074x; 1.0051x over previous
import functools

import jax
import jax.numpy as jnp
from jax import lax
from jax.experimental import pallas as pl
from jax.experimental.pallas import tpu as pltpu
from jax.experimental.pallas import tpu_sc as plsc

_NUM_CORES = 2
_NUM_SUBCORES = 16
_NUM_WORKERS = _NUM_CORES * _NUM_SUBCORES
_LANES = 16


_ROWS_PER_GROUP = 8


def _sc_pool_fn(batch, ctx, vocab, embed):
  bpw = batch // _NUM_WORKERS
  rpg = _ROWS_PER_GROUP
  gpw = bpw // rpg
  ipg = rpg * ctx
  lanes_per_row = embed // _LANES
  inv_ctx = 1.0 / ctx

  def body(table_hbm, ids_hbm, out_hbm, raw_v, rows_a, rows_b, acc_v,
           sem_a, sem_b):
    wid = lax.axis_index("s") * _NUM_CORES + lax.axis_index("c")
    pltpu.sync_copy(ids_hbm.at[pl.ds(wid * gpw, gpw)], raw_v)

    bufs = (rows_a, rows_b)
    sems = (sem_a, sem_b)

    def fire(g, b):
      return pltpu.async_copy(table_hbm.at[raw_v.at[g]], bufs[b], sems[b])

    def drain(b):
      pltpu.make_async_copy(
          table_hbm.at[pl.ds(0, ipg)], bufs[b], sems[b]).wait()

    fire(0, 0)
    fire(1, 1)

    def pair_body(i, carry):
      for b in range(2):
        g = 2 * i + b
        drain(b)
        buf = bufs[b]
        for j in range(rpg):
          for cc in range(lanes_per_row):
            s = buf[j * ctx, pl.ds(cc * _LANES, _LANES)]
            for t in range(1, ctx):
              s = s + buf[j * ctx + t, pl.ds(cc * _LANES, _LANES)]
            acc_v[g * rpg + j, pl.ds(cc * _LANES, _LANES)] = s * inv_ctx

        @pl.when(g + 2 < gpw)
        def _():
          fire(g + 2, b)
      return carry

    lax.fori_loop(0, gpw // 2, pair_body, 0)
    pltpu.sync_copy(acc_v, out_hbm.at[pl.ds(wid * bpw, bpw)])

  return pl.kernel(
      body,
      out_type=jax.ShapeDtypeStruct((batch, embed), jnp.float32),
      mesh=plsc.VectorSubcoreMesh(core_axis_name="c", subcore_axis_name="s"),
      compiler_params=pltpu.CompilerParams(use_tc_tiling_on_sc=False),
      scratch_types=[
          pltpu.VMEM((gpw, ipg), jnp.int32),
          pltpu.VMEM((ipg, embed), jnp.float32),
          pltpu.VMEM((ipg, embed), jnp.float32),
          pltpu.VMEM((bpw, embed), jnp.float32),
          pltpu.SemaphoreType.DMA,
          pltpu.SemaphoreType.DMA,
      ],
  )


def _mm_body(p_ref, w_ref, b_ref, o_ref):
  o_ref[...] = lax.dot_general(
      p_ref[...], w_ref[...],
      dimension_numbers=(((1,), (1,)), ((), ())),
      preferred_element_type=jnp.float32) + b_ref[...]


def _mm_fn(batch, vocab, embed, n_tile):
  grid = (pl.cdiv(vocab, n_tile),)
  return pl.pallas_call(
      _mm_body,
      grid=grid,
      in_specs=[
          pl.BlockSpec((batch, embed), lambda i: (0, 0)),
          pl.BlockSpec((n_tile, embed), lambda i: (i, 0)),
          pl.BlockSpec((1, n_tile), lambda i: (0, i)),
      ],
      out_specs=pl.BlockSpec((batch, n_tile), lambda i: (0, i)),
      out_shape=jax.ShapeDtypeStruct((batch, vocab), jnp.float32),
  )


@functools.lru_cache(maxsize=None)
def _build(batch, ctx, vocab, embed):
  return _sc_pool_fn(batch, ctx, vocab, embed), _mm_fn(batch, vocab, embed, 1024)


def kernel(context_ids, embed_table, fc_weight, fc_bias):
  batch, ctx = context_ids.shape
  vocab, embed = embed_table.shape
  sc_pool, mm = _build(batch, ctx, vocab, embed)
  ids = context_ids.astype(jnp.int32).reshape(
      batch // _ROWS_PER_GROUP, _ROWS_PER_GROUP * ctx)
  pooled = sc_pool(embed_table, ids)
  return mm(pooled, fc_weight, fc_bias.reshape(1, vocab))


# speedup vs baseline: 2.9973x; 2.9753x over previous
import functools

import jax
import jax.numpy as jnp
from jax import lax
from jax.experimental import pallas as pl
from jax.experimental.pallas import tpu as pltpu
from jax.experimental.pallas import tpu_sc as plsc

_NUM_CORES = 2
_NUM_SUBCORES = 16
_NUM_WORKERS = _NUM_CORES * _NUM_SUBCORES
_LANES = 16


_ROWS_PER_GROUP = 8


def _sc_pool_fn(batch, ctx, vocab, embed):
  bpw = batch // _NUM_WORKERS
  rpg = _ROWS_PER_GROUP
  gpw = bpw // rpg
  ipg = rpg * ctx
  lanes_per_row = embed // _LANES
  inv_ctx = 1.0 / ctx

  def body(table_hbm, ids_hbm, out_hbm, raw_v, rows_a, rows_b, acc_v,
           sem_a, sem_b):
    wid = lax.axis_index("s") * _NUM_CORES + lax.axis_index("c")
    pltpu.sync_copy(ids_hbm.at[pl.ds(wid * gpw, gpw)], raw_v)

    bufs = (rows_a, rows_b)
    sems = (sem_a, sem_b)

    def fire(g, b):
      return pltpu.async_copy(table_hbm.at[raw_v.at[g]], bufs[b], sems[b])

    def drain(b):
      pltpu.make_async_copy(
          table_hbm.at[pl.ds(0, ipg)], bufs[b], sems[b]).wait()

    fire(0, 0)
    fire(1, 1)

    def pair_body(i, carry):
      for b in range(2):
        g = 2 * i + b
        drain(b)
        buf = bufs[b]
        for j in range(rpg):
          for cc in range(lanes_per_row):
            s = buf[j * ctx, pl.ds(cc * _LANES, _LANES)]
            for t in range(1, ctx):
              s = s + buf[j * ctx + t, pl.ds(cc * _LANES, _LANES)]
            acc_v[g * rpg + j, pl.ds(cc * _LANES, _LANES)] = s * inv_ctx

        @pl.when(g + 2 < gpw)
        def _():
          fire(g + 2, b)
      return carry

    lax.fori_loop(0, gpw // 2, pair_body, 0)
    pltpu.sync_copy(acc_v, out_hbm.at[pl.ds(wid * bpw, bpw)])

  return pl.kernel(
      body,
      out_type=jax.ShapeDtypeStruct((batch, embed), jnp.float32),
      mesh=plsc.VectorSubcoreMesh(core_axis_name="c", subcore_axis_name="s"),
      compiler_params=pltpu.CompilerParams(use_tc_tiling_on_sc=False),
      scratch_types=[
          pltpu.VMEM((gpw, ipg), jnp.int32),
          pltpu.VMEM((ipg, embed), jnp.float32),
          pltpu.VMEM((ipg, embed), jnp.float32),
          pltpu.VMEM((bpw, embed), jnp.float32),
          pltpu.SemaphoreType.DMA,
          pltpu.SemaphoreType.DMA,
      ],
  )


def _mm_body(p_ref, w_ref, b_ref, o_ref):
  o_ref[...] = lax.dot_general(
      w_ref[...], p_ref[...],
      dimension_numbers=(((1,), (1,)), ((), ())),
      preferred_element_type=jnp.float32) + b_ref[...]


def _mm_fn(batch, vocab, embed, n_tile):
  grid = (pl.cdiv(vocab, n_tile),)
  return pl.pallas_call(
      _mm_body,
      grid=grid,
      in_specs=[
          pl.BlockSpec((batch, embed), lambda i: (0, 0)),
          pl.BlockSpec((n_tile, embed), lambda i: (i, 0)),
          pl.BlockSpec((n_tile, 1), lambda i: (i, 0)),
      ],
      out_specs=pl.BlockSpec((n_tile, batch), lambda i: (i, 0)),
      out_shape=jax.ShapeDtypeStruct((vocab, batch), jnp.float32),
  )


@functools.lru_cache(maxsize=None)
def _build(batch, ctx, vocab, embed):
  return _sc_pool_fn(batch, ctx, vocab, embed), _mm_fn(batch, vocab, embed, 1024)


def kernel(context_ids, embed_table, fc_weight, fc_bias):
  batch, ctx = context_ids.shape
  vocab, embed = embed_table.shape
  sc_pool, mm = _build(batch, ctx, vocab, embed)
  ids = context_ids.astype(jnp.int32).reshape(
      batch // _ROWS_PER_GROUP, _ROWS_PER_GROUP * ctx)
  pooled = sc_pool(embed_table, ids)
  return mm(pooled, fc_weight, fc_bias.reshape(vocab, 1)).T


# speedup vs baseline: 3.3136x; 1.1055x over previous
import functools

import jax
import jax.numpy as jnp
from jax import lax
from jax.experimental import pallas as pl
from jax.experimental.pallas import tpu as pltpu
from jax.experimental.pallas import tpu_sc as plsc

_NUM_CORES = 2
_NUM_SUBCORES = 16
_NUM_WORKERS = _NUM_CORES * _NUM_SUBCORES
_LANES = 16


_ROWS_PER_GROUP = 8


def _sc_pool_fn(batch, ctx, vocab, embed):
  bpw = batch // _NUM_WORKERS
  rpg = _ROWS_PER_GROUP
  gpw = bpw // rpg
  ipg = rpg * ctx
  lanes_per_row = embed // _LANES
  inv_ctx = 1.0 / ctx

  def body(table_hbm, ids_hbm, out_hbm, raw_v, rows_a, rows_b, acc_v,
           sem_a, sem_b):
    wid = lax.axis_index("s") * _NUM_CORES + lax.axis_index("c")
    pltpu.sync_copy(ids_hbm.at[pl.ds(wid * gpw, gpw)], raw_v)

    bufs = (rows_a, rows_b)
    sems = (sem_a, sem_b)

    def fire(g, b):
      return pltpu.async_copy(table_hbm.at[raw_v.at[g]], bufs[b], sems[b])

    def drain(b):
      pltpu.make_async_copy(
          table_hbm.at[pl.ds(0, ipg)], bufs[b], sems[b]).wait()

    fire(0, 0)
    fire(1, 1)

    def pair_body(i, carry):
      for b in range(2):
        g = 2 * i + b
        drain(b)
        buf = bufs[b]
        for j in range(rpg):
          for cc in range(lanes_per_row):
            s = buf[j * ctx, pl.ds(cc * _LANES, _LANES)]
            for t in range(1, ctx):
              s = s + buf[j * ctx + t, pl.ds(cc * _LANES, _LANES)]
            acc_v[g * rpg + j, pl.ds(cc * _LANES, _LANES)] = s * inv_ctx

        @pl.when(g + 2 < gpw)
        def _():
          fire(g + 2, b)
      return carry

    lax.fori_loop(0, gpw // 2, pair_body, 0)
    pltpu.sync_copy(acc_v, out_hbm.at[pl.ds(wid * bpw, bpw)])

  return pl.kernel(
      body,
      out_type=jax.ShapeDtypeStruct((batch, embed), jnp.float32),
      mesh=plsc.VectorSubcoreMesh(core_axis_name="c", subcore_axis_name="s"),
      compiler_params=pltpu.CompilerParams(use_tc_tiling_on_sc=False),
      scratch_types=[
          pltpu.VMEM((gpw, ipg), jnp.int32),
          pltpu.VMEM((ipg, embed), jnp.float32),
          pltpu.VMEM((ipg, embed), jnp.float32),
          pltpu.VMEM((bpw, embed), jnp.float32),
          pltpu.SemaphoreType.DMA,
          pltpu.SemaphoreType.DMA,
      ],
  )


def _mm_body(p_ref, wt_ref, b_ref, o_ref):
  o_ref[...] = lax.dot_general(
      wt_ref[...], p_ref[...],
      dimension_numbers=(((0,), (1,)), ((), ())),
      preferred_element_type=jnp.float32) + b_ref[...][:, None]


def _mm_fn(batch, vocab, embed, n_tile):
  grid = (pl.cdiv(vocab, n_tile),)
  return pl.pallas_call(
      _mm_body,
      grid=grid,
      in_specs=[
          pl.BlockSpec((batch, embed), lambda i: (0, 0)),
          pl.BlockSpec((embed, n_tile), lambda i: (0, i)),
          pl.BlockSpec((n_tile,), lambda i: (i,)),
      ],
      out_specs=pl.BlockSpec((n_tile, batch), lambda i: (i, 0)),
      out_shape=jax.ShapeDtypeStruct((vocab, batch), jnp.float32),
  )


@functools.lru_cache(maxsize=None)
def _build(batch, ctx, vocab, embed):
  return _sc_pool_fn(batch, ctx, vocab, embed), _mm_fn(batch, vocab, embed, 1024)


def kernel(context_ids, embed_table, fc_weight, fc_bias):
  batch, ctx = context_ids.shape
  vocab, embed = embed_table.shape
  sc_pool, mm = _build(batch, ctx, vocab, embed)
  ids = context_ids.astype(jnp.int32).reshape(
      batch // _ROWS_PER_GROUP, _ROWS_PER_GROUP * ctx)
  pooled = sc_pool(embed_table, ids)
  return mm(pooled, fc_weight.T, fc_bias).T
